# baseline (speedup 1.0000x reference)
import functools

import jax
import jax.numpy as jnp
from jax import lax
from jax.experimental import pallas as pl
from jax.experimental.pallas import tpu as pltpu
from jax.experimental.pallas import tpu_sc as plsc

_K = 16
_RB = 256
_NW = 32


def _dist_tau_kernel(x_ref, y_ref, dist_ref, tau_ref):
    x = x_ref[0]
    y = y_ref[0]
    n = y.shape[-1]
    xn = x / jnp.maximum(jnp.sqrt(jnp.sum(x * x, axis=0, keepdims=True)), 1e-12)
    yn = y / jnp.maximum(jnp.sqrt(jnp.sum(y * y, axis=0, keepdims=True)), 1e-12)
    x2 = jnp.sum(xn * xn, axis=0)
    y2 = jnp.sum(yn * yn, axis=0)
    inner = jax.lax.dot_general(
        xn.astype(jnp.bfloat16), yn.astype(jnp.bfloat16),
        (((0,), (0,)), ((), ())),
        preferred_element_type=jnp.float32,
    )
    dist = (x2[:, None] + (-2.0) * inner) + y2[None, :]
    dist_ref[0] = dist
    g = jnp.min(dist.reshape(_RB, 16, n // 16), axis=2)
    tau_ref[0, 0] = jnp.max(g, axis=1)


def _sc_topk_kernel(dist_hbm, tau_hbm, out_hbm, rowbuf, taubuf, offbuf,
                    candbuf, outbuf):
    n = dist_hbm.shape[1]
    rows_w = dist_hbm.shape[0] // _NW
    nvreg = n // 16
    wid = lax.axis_index("s") * 2 + lax.axis_index("c")
    base = wid * rows_w
    pltpu.sync_copy(tau_hbm.at[pl.ds(base, rows_w)], taubuf)
    lane = lax.iota(jnp.int32, 16)
    inf = jnp.float32(jnp.inf)

    def row_body(r, _):
        pltpu.sync_copy(dist_hbm.at[base + r], rowbuf)
        tau_v = plsc.load_gather(taubuf, [jnp.full((16,), r, jnp.int32)])

        def cb_body(cb, tot):
            def j_body(j, acc):
                v = rowbuf[pl.ds((cb * 16 + j) * 16, 16)]
                pc = plsc.all_reduce_population_count(v <= tau_v)
                return jnp.where(lane == j, pc, acc)

            acc = lax.fori_loop(0, 16, j_body, jnp.zeros((16,), jnp.int32),
                                unroll=True)
            cum = plsc.cumsum(acc)
            offbuf[pl.ds(cb * 16, 16)] = cum - acc + tot
            return tot + jnp.full((16,), jnp.max(cum))

        totv = lax.fori_loop(0, 16, cb_body, jnp.zeros((16,), jnp.int32))
        n_c = jnp.max(totv)

        def g_body(g, _):
            v = rowbuf[pl.ds(g * 16, 16)]
            m = v <= tau_v
            mi = m.astype(jnp.int32)
            off_v = plsc.load_gather(offbuf, [jnp.full((16,), g, jnp.int32)])
            pos = off_v + (plsc.cumsum(mi) - mi)
            plsc.store_scatter(candbuf, [pos], lane + g * 16, mask=m)
            return 0

        lax.fori_loop(0, nvreg, g_body, 0, unroll=4)

        def merge_body(t, carry):
            rk, ri = carry
            idxv = candbuf[pl.ds(t * 16, 16)]
            valid = (lane + t * 16) < n_c
            vals = plsc.load_gather(rowbuf, [jnp.where(valid, idxv, 0)])
            ck, ci = plsc.sort_key_val(
                jnp.where(valid, vals, inf),
                jnp.where(valid, idxv, jnp.int32(2**30)),
            )
            bk = lax.rev(rk, (0,))
            bi = lax.rev(ri, (0,))
            take_a = (ck < bk) | ((ck == bk) & (ci < bi))
            return tuple(plsc.sort_key_val(
                jnp.where(take_a, ck, bk), jnp.where(take_a, ci, bi)
            ))

        rk0 = jnp.full((16,), inf)
        ri0 = jnp.full((16,), 2**30, dtype=jnp.int32)
        _, ri = lax.fori_loop(0, (n_c + 15) // 16, merge_body, (rk0, ri0))
        outbuf[r] = ri
        return 0

    lax.fori_loop(0, rows_w, row_body, 0)
    pltpu.sync_copy(outbuf, out_hbm.at[pl.ds(base, rows_w)])


def kernel(x, y):
    b, c, n, _ = x.shape
    xs = x[..., 0]
    ys = y[..., 0]
    dist, tau = pl.pallas_call(
        _dist_tau_kernel,
        grid=(b, n // _RB),
        in_specs=[
            pl.BlockSpec((1, c, _RB), lambda bi, i: (bi, 0, i)),
            pl.BlockSpec((1, c, n), lambda bi, i: (bi, 0, 0)),
        ],
        out_specs=[
            pl.BlockSpec((1, _RB, n), lambda bi, i: (bi, i, 0)),
            pl.BlockSpec((1, 1, _RB), lambda bi, i: (bi * (n // _RB) + i, 0, 0)),
        ],
        out_shape=[
            jax.ShapeDtypeStruct((b, n, n), jnp.float32),
            jax.ShapeDtypeStruct((b * (n // _RB), 1, _RB), jnp.float32),
        ],
    )(xs, ys)

    rows = b * n
    rows_w = rows // _NW
    sc_topk = functools.partial(
        pl.kernel,
        out_type=jax.ShapeDtypeStruct((rows, _K), jnp.int32),
        mesh=plsc.VectorSubcoreMesh(core_axis_name="c", subcore_axis_name="s"),
        compiler_params=pltpu.CompilerParams(needs_layout_passes=False),
        scratch_types=[
            pltpu.VMEM((n,), jnp.float32),
            pltpu.VMEM((rows_w,), jnp.float32),
            pltpu.VMEM((n // 16,), jnp.int32),
            pltpu.VMEM((n + 16,), jnp.int32),
            pltpu.VMEM((rows_w, _K), jnp.int32),
        ],
    )(_sc_topk_kernel)
    nn_idx = sc_topk(dist.reshape(rows, n), tau.reshape(rows)).reshape(b, n, _K)

    center_idx = jnp.broadcast_to(
        jnp.arange(n, dtype=jnp.int32)[None, :, None], (b, n, _K)
    )
    return jnp.stack((nn_idx, center_idx), axis=0)


# speedup vs baseline: 1.4461x; 1.4461x over previous
import functools

import jax
import jax.numpy as jnp
from jax import lax
from jax.experimental import pallas as pl
from jax.experimental.pallas import tpu as pltpu
from jax.experimental.pallas import tpu_sc as plsc

_K = 16
_KB = 256
_NW = 32
_CH = 1024
_CAP = 768


def _dist_tau_kernel(x_ref, y_ref, dist_ref, tau_ref):
    x = x_ref[0]
    y = y_ref[0]
    xn = x / jnp.maximum(jnp.sqrt(jnp.sum(x * x, axis=0, keepdims=True)), 1e-12)
    yn = y / jnp.maximum(jnp.sqrt(jnp.sum(y * y, axis=0, keepdims=True)), 1e-12)
    x2 = jnp.sum(xn * xn, axis=0)
    y2 = jnp.sum(yn * yn, axis=0)
    inner = jax.lax.dot_general(
        yn.astype(jnp.bfloat16), xn.astype(jnp.bfloat16),
        (((0,), (0,)), ((), ())),
        preferred_element_type=jnp.float32,
    )
    dist = (x2[None, :] + (-2.0) * inner) + y2[:, None]
    dist_ref[0] = dist
    tau_ref[0, 0, 0] = jnp.min(dist, axis=0)


def _sc_topk_kernel(dist_hbm, taup_hbm, out_hbm, chunkbuf, taubuf, cval, cidx,
                    outbuf):
    nb, n, _ = dist_hbm.shape
    q_w = 256
    ng = q_w // 16
    nw_b = n // q_w
    wid = lax.axis_index("s") * 2 + lax.axis_index("c")
    b = wid // nw_b
    qbase = (wid % nw_b) * q_w
    pltpu.sync_copy(taup_hbm.at[b, :, 0, pl.ds(qbase, q_w)], taubuf)
    lane = lax.iota(jnp.int32, 16)
    inf = jnp.float32(jnp.inf)

    def group_body(g, _):
        tau_v = taubuf[0, pl.ds(g * 16, 16)]
        for kg in range(1, 16):
            tau_v = jnp.maximum(tau_v, taubuf[kg, pl.ds(g * 16, 16)])

        def chunk_body(kc, carry):
            cnt, jv = carry
            pltpu.sync_copy(
                dist_hbm.at[b, pl.ds(kc * _CH, _CH), pl.ds(qbase + g * 16, 16)],
                chunkbuf,
            )

            def col_body(j, carry2):
                cnt2, jv2 = carry2
                v = chunkbuf[j]
                m = v <= tau_v
                pos = jnp.minimum(cnt2, _CAP - 1)
                plsc.store_scatter(cval, [lane, pos], v, mask=m)
                plsc.store_scatter(cidx, [lane, pos], jv2, mask=m)
                return cnt2 + m.astype(jnp.int32), jv2 + 1

            return lax.fori_loop(0, _CH, col_body, (cnt, jv), unroll=8)

        cnt0 = jnp.zeros((16,), jnp.int32)
        cnt, _ = lax.fori_loop(0, n // _CH, chunk_body,
                               (cnt0, jnp.zeros((16,), jnp.int32)))

        for rr in range(16):
            n_c = cnt[rr]

            def merge_body(t, carry, rr=rr):
                rk, ri = carry
                valid = (lane + t * 16) < n_c
                vals = cval[rr, pl.ds(t * 16, 16)]
                idxv = cidx[rr, pl.ds(t * 16, 16)]
                ck, ci = plsc.sort_key_val(
                    jnp.where(valid, vals, inf),
                    jnp.where(valid, idxv, jnp.int32(2**30)),
                )
                bk = lax.rev(rk, (0,))
                bi = lax.rev(ri, (0,))
                take_a = (ck < bk) | ((ck == bk) & (ci < bi))
                return tuple(plsc.sort_key_val(
                    jnp.where(take_a, ck, bk), jnp.where(take_a, ci, bi)
                ))

            rk0 = jnp.full((16,), inf)
            ri0 = jnp.full((16,), 2**30, dtype=jnp.int32)
            _, ri = lax.fori_loop(0, (n_c + 15) // 16, merge_body, (rk0, ri0))
            outbuf[g * 16 + rr] = ri
        return 0

    lax.fori_loop(0, ng, group_body, 0)
    pltpu.sync_copy(outbuf, out_hbm.at[pl.ds(wid * q_w, q_w)])


def kernel(x, y):
    b, c, n, _ = x.shape
    xs = x[..., 0]
    ys = y[..., 0]
    dist_t, tau_p = pl.pallas_call(
        _dist_tau_kernel,
        grid=(b, n // _KB),
        in_specs=[
            pl.BlockSpec((1, c, n), lambda bi, i: (bi, 0, 0)),
            pl.BlockSpec((1, c, _KB), lambda bi, i: (bi, 0, i)),
        ],
        out_specs=[
            pl.BlockSpec((1, _KB, n), lambda bi, i: (bi, i, 0)),
            pl.BlockSpec((1, 1, 1, n), lambda bi, i: (bi, i, 0, 0)),
        ],
        out_shape=[
            jax.ShapeDtypeStruct((b, n, n), jnp.float32),
            jax.ShapeDtypeStruct((b, n // _KB, 1, n), jnp.float32),
        ],
    )(xs, ys)

    rows = b * n
    sc_topk = functools.partial(
        pl.kernel,
        out_type=jax.ShapeDtypeStruct((rows, _K), jnp.int32),
        mesh=plsc.VectorSubcoreMesh(core_axis_name="c", subcore_axis_name="s"),
        compiler_params=pltpu.CompilerParams(
            needs_layout_passes=False, use_tc_tiling_on_sc=False
        ),
        scratch_types=[
            pltpu.VMEM((_CH, 16), jnp.float32),
            pltpu.VMEM((n // _KB, rows // _NW), jnp.float32),
            pltpu.VMEM((16, _CAP), jnp.float32),
            pltpu.VMEM((16, _CAP), jnp.int32),
            pltpu.VMEM((rows // _NW, _K), jnp.int32),
        ],
    )(_sc_topk_kernel)
    nn_idx = sc_topk(dist_t, tau_p).reshape(b, n, _K)

    center_idx = jnp.broadcast_to(
        jnp.arange(n, dtype=jnp.int32)[None, :, None], (b, n, _K)
    )
    return jnp.stack((nn_idx, center_idx), axis=0)


# speedup vs baseline: 1.6815x; 1.1628x over previous
import functools

import jax
import jax.numpy as jnp
from jax import lax
from jax.experimental import pallas as pl
from jax.experimental.pallas import tpu as pltpu
from jax.experimental.pallas import tpu_sc as plsc

_K = 16
_KB = 256
_NW = 32
_CH = 1024
_CAP = 512


def _dist_tau_kernel(x_ref, y_ref, dist_ref, tau_ref):
    x = x_ref[0]
    y = y_ref[0]
    xn = x / jnp.maximum(jnp.sqrt(jnp.sum(x * x, axis=0, keepdims=True)), 1e-12)
    yn = y / jnp.maximum(jnp.sqrt(jnp.sum(y * y, axis=0, keepdims=True)), 1e-12)
    x2 = jnp.sum(xn * xn, axis=0)
    y2 = jnp.sum(yn * yn, axis=0)
    inner = jax.lax.dot_general(
        yn.astype(jnp.bfloat16), xn.astype(jnp.bfloat16),
        (((0,), (0,)), ((), ())),
        preferred_element_type=jnp.float32,
    )
    dist = (x2[None, :] + (-2.0) * inner) + y2[:, None]
    dist_ref[0] = dist
    tau_ref[0, 0, 0] = jnp.min(dist, axis=0)


def _sc_topk_kernel(dist_hbm, taup_hbm, out_hbm, chunk0, chunk1, taubuf, cval,
                    cidx, outbuf, sem0, sem1):
    nb, n, _ = dist_hbm.shape
    q_w = 256
    ng = q_w // 16
    nch = n // _CH
    nw_b = n // q_w
    wid = lax.axis_index("s") * 2 + lax.axis_index("c")
    b = wid // nw_b
    qbase = (wid % nw_b) * q_w
    pltpu.sync_copy(taup_hbm.at[b, :, 0, pl.ds(qbase, q_w)], taubuf)
    lane = lax.iota(jnp.int32, 16)
    inf = jnp.float32(jnp.inf)
    bufs = (chunk0, chunk1)
    sems = (sem0, sem1)

    def start_fetch(g, kc, buf, sem):
        pltpu.async_copy(
            dist_hbm.at[b, pl.ds(kc * _CH, _CH), pl.ds(qbase + g * 16, 16)],
            buf, sem,
        )

    def wait_fetch(buf, sem):
        pltpu.make_async_copy(dist_hbm.at[b, pl.ds(0, _CH), pl.ds(0, 16)],
                              buf, sem).wait()

    start_fetch(0, 0, bufs[0], sems[0])

    def group_body(g, _):
        tau_v = taubuf[0, pl.ds(g * 16, 16)]
        for kg in range(1, 16):
            tau_v = jnp.maximum(tau_v, taubuf[kg, pl.ds(g * 16, 16)])

        cnt = jnp.zeros((16,), jnp.int32)
        jv = jnp.zeros((16,), jnp.int32)
        for kc in range(nch):
            wait_fetch(bufs[kc % 2], sems[kc % 2])
            if kc < nch - 1:
                start_fetch(g, kc + 1, bufs[(kc + 1) % 2], sems[(kc + 1) % 2])
            else:
                @pl.when(g < ng - 1)
                def _():
                    start_fetch(g + 1, 0, bufs[0], sems[0])

            def col_body(j, carry, kc=kc):
                cnt2, jv2 = carry
                v = bufs[kc % 2][j]
                m = v <= tau_v
                pos = cnt2 & (_CAP - 1)
                plsc.store_scatter(cval, [lane, pos], v, mask=m)
                plsc.store_scatter(cidx, [lane, pos], jv2, mask=m)
                return cnt2 + m.astype(jnp.int32), jv2 + 1

            cnt, jv = lax.fori_loop(0, _CH, col_body, (cnt, jv), unroll=8)

        for rr in range(16):
            n_c = jnp.minimum(cnt[rr], _CAP)

            def merge_body(t, carry, rr=rr):
                rk, ri = carry
                valid = (lane + t * 16) < n_c
                vals = cval[rr, pl.ds(t * 16, 16)]
                idxv = cidx[rr, pl.ds(t * 16, 16)]
                ck, ci = plsc.sort_key_val(
                    jnp.where(valid, vals, inf),
                    jnp.where(valid, idxv, jnp.int32(2**30)),
                )
                bk = lax.rev(rk, (0,))
                bi = lax.rev(ri, (0,))
                take_a = (ck < bk) | ((ck == bk) & (ci < bi))
                return tuple(plsc.sort_key_val(
                    jnp.where(take_a, ck, bk), jnp.where(take_a, ci, bi)
                ))

            rk0 = jnp.full((16,), inf)
            ri0 = jnp.full((16,), 2**30, dtype=jnp.int32)
            _, ri = lax.fori_loop(0, (n_c + 15) // 16, merge_body, (rk0, ri0))
            outbuf[g * 16 + rr] = ri
        return 0

    lax.fori_loop(0, ng, group_body, 0)
    pltpu.sync_copy(outbuf, out_hbm.at[pl.ds(wid * q_w, q_w)])


def kernel(x, y):
    b, c, n, _ = x.shape
    xs = x[..., 0]
    ys = y[..., 0]
    dist_t, tau_p = pl.pallas_call(
        _dist_tau_kernel,
        grid=(b, n // _KB),
        in_specs=[
            pl.BlockSpec((1, c, n), lambda bi, i: (bi, 0, 0)),
            pl.BlockSpec((1, c, _KB), lambda bi, i: (bi, 0, i)),
        ],
        out_specs=[
            pl.BlockSpec((1, _KB, n), lambda bi, i: (bi, i, 0)),
            pl.BlockSpec((1, 1, 1, n), lambda bi, i: (bi, i, 0, 0)),
        ],
        out_shape=[
            jax.ShapeDtypeStruct((b, n, n), jnp.float32),
            jax.ShapeDtypeStruct((b, n // _KB, 1, n), jnp.float32),
        ],
    )(xs, ys)

    rows = b * n
    sc_topk = functools.partial(
        pl.kernel,
        out_type=jax.ShapeDtypeStruct((rows, _K), jnp.int32),
        mesh=plsc.VectorSubcoreMesh(core_axis_name="c", subcore_axis_name="s"),
        compiler_params=pltpu.CompilerParams(
            needs_layout_passes=False, use_tc_tiling_on_sc=False
        ),
        scratch_types=[
            pltpu.VMEM((_CH, 16), jnp.float32),
            pltpu.VMEM((_CH, 16), jnp.float32),
            pltpu.VMEM((n // _KB, rows // _NW), jnp.float32),
            pltpu.VMEM((16, _CAP), jnp.float32),
            pltpu.VMEM((16, _CAP), jnp.int32),
            pltpu.VMEM((rows // _NW, _K), jnp.int32),
            pltpu.SemaphoreType.DMA,
            pltpu.SemaphoreType.DMA,
        ],
    )(_sc_topk_kernel)
    nn_idx = sc_topk(dist_t, tau_p).reshape(b, n, _K)

    center_idx = jnp.broadcast_to(
        jnp.arange(n, dtype=jnp.int32)[None, :, None], (b, n, _K)
    )
    return jnp.stack((nn_idx, center_idx), axis=0)
